# baseline (speedup 1.0000x reference)
import functools

import jax
import jax.numpy as jnp
from jax.experimental import pallas as pl
from jax.experimental.pallas import tpu as pltpu

N = 10000
E = 320000
D = 128
HID = 128

EBLK = 512


def _dotf(a, b):
    return jax.lax.dot_general(a, b, (((1,), (0,)), ((), ())),
                               preferred_element_type=jnp.float32)


def _silu(x):
    return x * jax.nn.sigmoid(x)


def _edge_body(need_coord, g1_ref, g2_ref, rad_ref, we1_ref, be1_ref,
               we2_ref, be2_ref, wc1_ref, bc1_ref, wc2_ref,
               m_ref, cw_ref, s1_ref):
    radial = rad_ref[...][:, :1]
    f = jnp.concatenate([g1_ref[...], g2_ref[...], radial], axis=1)
    s1_ref[...] = _dotf(f, we1_ref[...])
    m = _silu(s1_ref[...] + be1_ref[...])
    s1_ref[...] = _dotf(m, we2_ref[...])
    m = _silu(s1_ref[...] + be2_ref[...])
    m_ref[...] = m
    if need_coord:
        s1_ref[...] = _dotf(m, wc1_ref[...])
        c = _silu(s1_ref[...] + bc1_ref[...])
        cw = _dotf(c, wc2_ref[...])
        cw_ref[...] = jnp.concatenate(
            [cw, jnp.zeros((EBLK, 7), jnp.float32)], axis=1)
    else:
        cw_ref[...] = jnp.zeros((EBLK, 8), jnp.float32)


def _edge_mlp(g1, g2, rad8, p, need_coord):
    body = functools.partial(_edge_body, need_coord)
    return pl.pallas_call(
        body,
        grid=(E // EBLK,),
        in_specs=[
            pl.BlockSpec((EBLK, D), lambda i: (i, 0)),
            pl.BlockSpec((EBLK, D), lambda i: (i, 0)),
            pl.BlockSpec((EBLK, 8), lambda i: (i, 0)),
            pl.BlockSpec((2 * D + 1, HID), lambda i: (0, 0)),
            pl.BlockSpec((1, HID), lambda i: (0, 0)),
            pl.BlockSpec((HID, HID), lambda i: (0, 0)),
            pl.BlockSpec((1, HID), lambda i: (0, 0)),
            pl.BlockSpec((HID, HID), lambda i: (0, 0)),
            pl.BlockSpec((1, HID), lambda i: (0, 0)),
            pl.BlockSpec((HID, 1), lambda i: (0, 0)),
        ],
        out_specs=[pl.BlockSpec((EBLK, HID), lambda i: (i, 0)),
                   pl.BlockSpec((EBLK, 8), lambda i: (i, 0))],
        out_shape=[jax.ShapeDtypeStruct((E, HID), jnp.float32),
                   jax.ShapeDtypeStruct((E, 8), jnp.float32)],
        scratch_shapes=[pltpu.VMEM((EBLK, HID), jnp.float32)],
    )(g1, g2, rad8, p['We1'], p['be1'].reshape(1, -1),
      p['We2'], p['be2'].reshape(1, -1), p['Wc1'], p['bc1'].reshape(1, -1),
      p['Wc2'])


def _layer(p, h, x, src, dst, need_coord):
    x_diff = x[src] - x[dst]
    radial = jnp.sum(x_diff * x_diff, axis=1, keepdims=True)
    rad8 = jnp.pad(radial, ((0, 0), (0, 7)))
    g1 = h[src]
    g2 = h[dst]
    m, cw8 = _edge_mlp(g1, g2, rad8, p, need_coord)
    hacc = jax.ops.segment_sum(m, dst, num_segments=N)
    if need_coord:
        x_diffn = x_diff / (jnp.sqrt(radial) + 1e-30)
        msg_x = cw8[:, :1] * x_diffn
        x_sum = jax.ops.segment_sum(msg_x, dst, num_segments=N)
        deg = jax.ops.segment_sum(jnp.ones((E, 1), jnp.float32), dst,
                                  num_segments=N)
        x_new = x + x_sum / jnp.maximum(deg, 1.0)
    else:
        x_new = x
    hh = jnp.concatenate([h, hacc], axis=-1)
    h_new = jnp.dot(_silu(jnp.dot(hh, p['Wn1']) + p['bn1']),
                    p['Wn2']) + p['bn2']
    return h_new, x_new


def _bn_relu(h, g, b):
    mu = jnp.mean(h, axis=0)
    var = jnp.var(h, axis=0)
    return jax.nn.relu(g * (h - mu) / jnp.sqrt(var + 1e-5) + b)


def kernel(in_feat, coord, edge_index, params):
    src = edge_index[0]
    dst = edge_index[1]
    h, c = _layer(params['conv1'], in_feat, coord, src, dst, True)
    h = _bn_relu(h, params['bn1_g'], params['bn1_b'])
    h, c = _layer(params['conv2'], h, c, src, dst, True)
    h = _bn_relu(h, params['bn2_g'], params['bn2_b'])
    h, c = _layer(params['conv3'], h, c, src, dst, True)
    h = _bn_relu(h, params['bn3_g'], params['bn3_b'])
    h, _ = _layer(params['conv4'], h, c, src, dst, False)
    h = _bn_relu(h, params['bn4_g'], params['bn4_b'])
    return (h, c)


# speedup vs baseline: 1.2627x; 1.2627x over previous
import functools

import jax
import jax.numpy as jnp
from jax.experimental import pallas as pl
from jax.experimental.pallas import tpu as pltpu
from jax.experimental.pallas import tpu_sc as plsc

N = 10000
E = 320000
D = 128
HID = 128

EBLK = 512
GW = 128


def _sc_gather2(table, src_idx, dst_idx):
    mesh = plsc.VectorSubcoreMesh(core_axis_name="c", subcore_axis_name="s")

    @functools.partial(
        pl.kernel, mesh=mesh,
        out_type=[jax.ShapeDtypeStruct((E, D), jnp.float32),
                  jax.ShapeDtypeStruct((E, D), jnp.float32)])
    def k(tab_hbm, si_hbm, di_hbm, g1_hbm, g2_hbm):
        def body(si_vmem, di_vmem, g1_vmem, g2_vmem):
            pltpu.sync_copy(tab_hbm.at[si_vmem.at[0]], g1_vmem)
            pltpu.sync_copy(tab_hbm.at[di_vmem.at[0]], g2_vmem)

        pltpu.emit_pipeline(
            body,
            grid=(E // GW,),
            in_specs=[pl.BlockSpec((1, GW), lambda i: (i, 0)),
                      pl.BlockSpec((1, GW), lambda i: (i, 0))],
            out_specs=[pl.BlockSpec((GW, D), lambda i: (i, 0)),
                       pl.BlockSpec((GW, D), lambda i: (i, 0))],
            core_axis_name=("c", "s"),
            dimension_semantics=(pltpu.PARALLEL,),
        )(si_hbm, di_hbm, g1_hbm, g2_hbm)

    return k(table, src_idx, dst_idx)


def _dotf(a, b):
    return jax.lax.dot_general(a, b, (((1,), (0,)), ((), ())),
                               preferred_element_type=jnp.float32)


def _silu(x):
    return x * jax.nn.sigmoid(x)


def _edge_body(need_coord, g1_ref, g2_ref, rad_ref, we1_ref, be1_ref,
               we2_ref, be2_ref, wc1_ref, bc1_ref, wc2_ref,
               m_ref, cw_ref, s1_ref):
    radial = rad_ref[...][:, :1]
    f = jnp.concatenate([g1_ref[...], g2_ref[...], radial], axis=1)
    s1_ref[...] = _dotf(f, we1_ref[...])
    m = _silu(s1_ref[...] + be1_ref[...])
    s1_ref[...] = _dotf(m, we2_ref[...])
    m = _silu(s1_ref[...] + be2_ref[...])
    m_ref[...] = m
    if need_coord:
        s1_ref[...] = _dotf(m, wc1_ref[...])
        c = _silu(s1_ref[...] + bc1_ref[...])
        cw = _dotf(c, wc2_ref[...])
        cw_ref[...] = jnp.concatenate(
            [cw, jnp.zeros((EBLK, 7), jnp.float32)], axis=1)
    else:
        cw_ref[...] = jnp.zeros((EBLK, 8), jnp.float32)


def _edge_mlp(g1, g2, rad8, p, need_coord):
    body = functools.partial(_edge_body, need_coord)
    return pl.pallas_call(
        body,
        grid=(E // EBLK,),
        in_specs=[
            pl.BlockSpec((EBLK, D), lambda i: (i, 0)),
            pl.BlockSpec((EBLK, D), lambda i: (i, 0)),
            pl.BlockSpec((EBLK, 8), lambda i: (i, 0)),
            pl.BlockSpec((2 * D + 1, HID), lambda i: (0, 0)),
            pl.BlockSpec((1, HID), lambda i: (0, 0)),
            pl.BlockSpec((HID, HID), lambda i: (0, 0)),
            pl.BlockSpec((1, HID), lambda i: (0, 0)),
            pl.BlockSpec((HID, HID), lambda i: (0, 0)),
            pl.BlockSpec((1, HID), lambda i: (0, 0)),
            pl.BlockSpec((HID, 1), lambda i: (0, 0)),
        ],
        out_specs=[pl.BlockSpec((EBLK, HID), lambda i: (i, 0)),
                   pl.BlockSpec((EBLK, 8), lambda i: (i, 0))],
        out_shape=[jax.ShapeDtypeStruct((E, HID), jnp.float32),
                   jax.ShapeDtypeStruct((E, 8), jnp.float32)],
        scratch_shapes=[pltpu.VMEM((EBLK, HID), jnp.float32)],
    )(g1, g2, rad8, p['We1'], p['be1'].reshape(1, -1),
      p['We2'], p['be2'].reshape(1, -1), p['Wc1'], p['bc1'].reshape(1, -1),
      p['Wc2'])


def _layer(p, h, x, src, dst, need_coord):
    x_diff = x[src] - x[dst]
    radial = jnp.sum(x_diff * x_diff, axis=1, keepdims=True)
    rad8 = jnp.pad(radial, ((0, 0), (0, 7)))
    g1, g2 = _sc_gather2(h, src.reshape(E // GW, GW), dst.reshape(E // GW, GW))
    m, cw8 = _edge_mlp(g1, g2, rad8, p, need_coord)
    hacc = jax.ops.segment_sum(m, dst, num_segments=N)
    if need_coord:
        x_diffn = x_diff / (jnp.sqrt(radial) + 1e-30)
        msg_x = cw8[:, :1] * x_diffn
        x_sum = jax.ops.segment_sum(msg_x, dst, num_segments=N)
        deg = jax.ops.segment_sum(jnp.ones((E, 1), jnp.float32), dst,
                                  num_segments=N)
        x_new = x + x_sum / jnp.maximum(deg, 1.0)
    else:
        x_new = x
    hh = jnp.concatenate([h, hacc], axis=-1)
    h_new = jnp.dot(_silu(jnp.dot(hh, p['Wn1']) + p['bn1']),
                    p['Wn2']) + p['bn2']
    return h_new, x_new


def _bn_relu(h, g, b):
    mu = jnp.mean(h, axis=0)
    var = jnp.var(h, axis=0)
    return jax.nn.relu(g * (h - mu) / jnp.sqrt(var + 1e-5) + b)


def kernel(in_feat, coord, edge_index, params):
    src = edge_index[0]
    dst = edge_index[1]
    h, c = _layer(params['conv1'], in_feat, coord, src, dst, True)
    h = _bn_relu(h, params['bn1_g'], params['bn1_b'])
    h, c = _layer(params['conv2'], h, c, src, dst, True)
    h = _bn_relu(h, params['bn2_g'], params['bn2_b'])
    h, c = _layer(params['conv3'], h, c, src, dst, True)
    h = _bn_relu(h, params['bn3_g'], params['bn3_b'])
    h, _ = _layer(params['conv4'], h, c, src, dst, False)
    h = _bn_relu(h, params['bn4_g'], params['bn4_b'])
    return (h, c)
